# Initial kernel scaffold; baseline (speedup 1.0000x reference)
import jax
import jax.numpy as jnp
from jax.experimental import pallas as pl


def kernel(x, embedding, W, b):
    raise NotImplementedError("write your pallas kernel here")


# speedup vs baseline: 8.8378x; 8.8378x over previous
import functools

import jax
import jax.numpy as jnp
from jax import lax
from jax.experimental import pallas as pl
from jax.experimental.pallas import tpu as pltpu
from jax.experimental.pallas import tpu_sc as plsc

_NC = 2
_NS = 16
_NW = _NC * _NS
_LANES = 16


def _project_table(embedding, W, b):
    V, E = embedding.shape
    D = W.shape[1]
    BLK = 2000
    assert V % BLK == 0

    def mm(e_ref, w_ref, b_ref, o_ref):
        o_ref[...] = (
            jnp.dot(e_ref[...], w_ref[...], preferred_element_type=jnp.float32)
            + b_ref[...]
        )

    return pl.pallas_call(
        mm,
        grid=(V // BLK,),
        in_specs=[
            pl.BlockSpec((BLK, E), lambda i: (i, 0)),
            pl.BlockSpec((E, D), lambda i: (0, 0)),
            pl.BlockSpec((1, D), lambda i: (0, 0)),
        ],
        out_specs=pl.BlockSpec((BLK, D), lambda i: (i, 0)),
        out_shape=jax.ShapeDtypeStruct((V, D), jnp.float32),
    )(embedding, W, b.reshape(1, D))


def _rsqrt16(t):
    i = lax.bitcast_convert_type(t, jnp.int32)
    i = jnp.int32(0x5F3759DF) - lax.shift_right_arithmetic(i, 1)
    y = lax.bitcast_convert_type(i, jnp.float32)
    half_t = 0.5 * t
    for _ in range(3):
        y = y * (1.5 - half_t * y * y)
    return y


def _gather_normalize(P, x_flat, L, D):
    N = x_flat.shape[0]
    PER_W = N // _NW
    BROWS_PER_G = 4
    GR = BROWS_PER_G * L
    NG = PER_W // GR
    NCH = D // _LANES
    SPLIT = 128
    assert PER_W % GR == 0 and GR > SPLIT and GR % 8 == 0 and SPLIT % 8 == 0

    mesh = plsc.VectorSubcoreMesh(core_axis_name="c", subcore_axis_name="s")

    @functools.partial(
        pl.kernel,
        mesh=mesh,
        out_type=jax.ShapeDtypeStruct((N, D), jnp.float32),
        scratch_types=[
            pltpu.VMEM((PER_W,), jnp.int32),
            pltpu.VMEM((GR, D), jnp.float32),
            pltpu.VMEM((GR, D), jnp.float32),
            pltpu.VMEM((GR, D), jnp.float32),
            pltpu.VMEM((GR, D), jnp.float32),
            pltpu.SemaphoreType.DMA,
            pltpu.SemaphoreType.DMA,
            pltpu.SemaphoreType.DMA,
            pltpu.SemaphoreType.DMA,
        ],
    )
    def sc_kernel(p_hbm, xf_hbm, o_hbm, idx_v, in0, in1, out0, out1, g0, g1, s0, s1):
        wid = lax.axis_index("s") * _NC + lax.axis_index("c")
        base = wid * PER_W
        ins = (in0, in1)
        outs = (out0, out1)
        gsems = (g0, g1)
        ssems = (s0, s1)

        pltpu.sync_copy(xf_hbm.at[pl.ds(base, PER_W)], idx_v)

        def gather_copies(g, k):
            off = g * GR
            return (
                pltpu.make_async_copy(
                    p_hbm.at[idx_v.at[pl.ds(off, SPLIT)]],
                    ins[k].at[pl.ds(0, SPLIT)],
                    gsems[k],
                ),
                pltpu.make_async_copy(
                    p_hbm.at[idx_v.at[pl.ds(off + SPLIT, GR - SPLIT)]],
                    ins[k].at[pl.ds(SPLIT, GR - SPLIT)],
                    gsems[k],
                ),
            )

        def scatter_copy(g, k):
            return pltpu.make_async_copy(
                outs[k], o_hbm.at[pl.ds(base + g * GR, GR)], ssems[k]
            )

        def issue_gather(g, k):
            for c in gather_copies(g, k):
                c.start()

        def wait_gather(g, k):
            for c in gather_copies(g, k):
                c.wait()

        def compute(k):
            ib, ob = ins[k], outs[k]
            for r in range(BROWS_PER_G):
                rb = r * L

                def acc_body(l, accs):
                    row = rb + l
                    new = []
                    for c in range(NCH):
                        v = ib[row, pl.ds(_LANES * c, _LANES)]
                        new.append(accs[c] + v * v)
                    return tuple(new)

                zeros = tuple(jnp.zeros((_LANES,), jnp.float32) for _ in range(NCH))
                accs = lax.fori_loop(0, L, acc_body, zeros)
                scales = tuple(
                    _rsqrt16(jnp.maximum(a, jnp.float32(1e-24))) for a in accs
                )

                def scale_body(l, carry):
                    row = rb + l
                    for c in range(NCH):
                        v = ib[row, pl.ds(_LANES * c, _LANES)] * scales[c]
                        ob[row, pl.ds(_LANES * c, _LANES)] = jnp.maximum(v, 0.0)
                    return carry

                lax.fori_loop(0, L, scale_body, 0)

        issue_gather(0, 0)
        issue_gather(1, 1)

        def outer(gp, carry):
            for k in range(2):
                g = 2 * gp + k
                wait_gather(g, k)

                @pl.when(gp >= 1)
                def _():
                    scatter_copy(g - 2, k).wait()

                compute(k)
                scatter_copy(g, k).start()

                @pl.when(gp < NG // 2 - 1)
                def _():
                    issue_gather(g + 2, k)
            return carry

        lax.fori_loop(0, NG // 2, outer, 0)
        scatter_copy(NG - 2, 0).wait()
        scatter_copy(NG - 1, 1).wait()

    return sc_kernel(P, x_flat)


def kernel(x, embedding, W, b):
    B, L = x.shape
    D = W.shape[1]
    P = _project_table(embedding, W, b)
    x_flat = x.reshape(B * L).astype(jnp.int32)
    out_flat = _gather_normalize(P, x_flat, L, D)
    return out_flat.reshape(B, L, D)


# speedup vs baseline: 11.0116x; 1.2460x over previous
import functools

import jax
import jax.numpy as jnp
from jax import lax
from jax.experimental import pallas as pl
from jax.experimental.pallas import tpu as pltpu
from jax.experimental.pallas import tpu_sc as plsc

_NC = 2
_NS = 16
_NW = _NC * _NS
_LANES = 16


def _project_table(embedding, W, b):
    V, E = embedding.shape
    D = W.shape[1]
    ET = embedding.T
    BLK = 2048
    grid = (V + BLK - 1) // BLK

    def mm(et_ref, w_ref, b_ref, o_ref):
        o_ref[...] = (
            jax.lax.dot_general(
                et_ref[...],
                w_ref[...],
                dimension_numbers=(((0,), (0,)), ((), ())),
                preferred_element_type=jnp.float32,
            )
            + b_ref[...]
        )

    return pl.pallas_call(
        mm,
        grid=(grid,),
        in_specs=[
            pl.BlockSpec((E, BLK), lambda i: (0, i)),
            pl.BlockSpec((E, D), lambda i: (0, 0)),
            pl.BlockSpec((1, D), lambda i: (0, 0)),
        ],
        out_specs=pl.BlockSpec((BLK, D), lambda i: (i, 0)),
        out_shape=jax.ShapeDtypeStruct((V, D), jnp.float32),
    )(ET, W, b.reshape(1, D))


def _rsqrt16(t):
    i = lax.bitcast_convert_type(t, jnp.int32)
    i = jnp.int32(0x5F3759DF) - lax.shift_right_arithmetic(i, 1)
    y = lax.bitcast_convert_type(i, jnp.float32)
    half_t = 0.5 * t
    for _ in range(3):
        y = y * (1.5 - half_t * y * y)
    return y


def _gather_normalize(P, x_flat, B, L, D):
    N = x_flat.shape[0]
    PER_W = N // _NW
    BPW = B // _NW
    BG = 4
    GR = BG * L
    NG = BPW // BG
    NCH = D // _LANES
    SPLIT = 128
    assert BPW % BG == 0 and GR > SPLIT and GR % 8 == 0

    mesh = plsc.VectorSubcoreMesh(core_axis_name="c", subcore_axis_name="s")

    @functools.partial(
        pl.kernel,
        mesh=mesh,
        out_type=jax.ShapeDtypeStruct((L, B, D), jnp.float32),
        scratch_types=[
            pltpu.VMEM((PER_W,), jnp.int32),
            pltpu.VMEM((GR, D), jnp.float32),
            pltpu.VMEM((GR, D), jnp.float32),
            pltpu.VMEM((L, BG, D), jnp.float32),
            pltpu.VMEM((L, BG, D), jnp.float32),
            pltpu.SemaphoreType.DMA,
            pltpu.SemaphoreType.DMA,
            pltpu.SemaphoreType.DMA,
            pltpu.SemaphoreType.DMA,
        ],
    )
    def sc_kernel(p_hbm, xf_hbm, o_hbm, idx_v, in0, in1, out0, out1, g0, g1, s0, s1):
        wid = lax.axis_index("s") * _NC + lax.axis_index("c")
        base = wid * PER_W
        bbase = wid * BPW
        ins = (in0, in1)
        outs = (out0, out1)
        gsems = (g0, g1)
        ssems = (s0, s1)

        pltpu.sync_copy(xf_hbm.at[pl.ds(base, PER_W)], idx_v)

        def gather_copies(g, k):
            off = g * GR
            return (
                pltpu.make_async_copy(
                    p_hbm.at[idx_v.at[pl.ds(off, SPLIT)]],
                    ins[k].at[pl.ds(0, SPLIT)],
                    gsems[k],
                ),
                pltpu.make_async_copy(
                    p_hbm.at[idx_v.at[pl.ds(off + SPLIT, GR - SPLIT)]],
                    ins[k].at[pl.ds(SPLIT, GR - SPLIT)],
                    gsems[k],
                ),
            )

        def scatter_copy(g, k):
            return pltpu.make_async_copy(
                outs[k],
                o_hbm.at[:, pl.ds(bbase + g * BG, BG), :],
                ssems[k],
            )

        def issue_gather(g, k):
            for c in gather_copies(g, k):
                c.start()

        def wait_gather(g, k):
            for c in gather_copies(g, k):
                c.wait()

        def compute(k):
            ib, ob = ins[k], outs[k]
            for r in range(BG):
                rb = r * L

                def acc_body(l, accs):
                    row = rb + l
                    new = []
                    for c in range(NCH):
                        v = ib[row, pl.ds(_LANES * c, _LANES)]
                        new.append(accs[c] + v * v)
                    return tuple(new)

                zeros = tuple(jnp.zeros((_LANES,), jnp.float32) for _ in range(NCH))
                accs = lax.fori_loop(0, L, acc_body, zeros)
                scales = tuple(
                    _rsqrt16(jnp.maximum(a, jnp.float32(1e-24))) for a in accs
                )

                def scale_body(l, carry):
                    row = rb + l
                    for c in range(NCH):
                        v = ib[row, pl.ds(_LANES * c, _LANES)] * scales[c]
                        ob[l, r, pl.ds(_LANES * c, _LANES)] = jnp.maximum(v, 0.0)
                    return carry

                lax.fori_loop(0, L, scale_body, 0)

        issue_gather(0, 0)
        issue_gather(1, 1)

        def outer(gp, carry):
            for k in range(2):
                g = 2 * gp + k
                wait_gather(g, k)

                @pl.when(gp >= 1)
                def _():
                    scatter_copy(g - 2, k).wait()

                compute(k)
                scatter_copy(g, k).start()

                @pl.when(gp < NG // 2 - 1)
                def _():
                    issue_gather(g + 2, k)
            return carry

        lax.fori_loop(0, NG // 2, outer, 0)
        scatter_copy(NG - 2, 0).wait()
        scatter_copy(NG - 1, 1).wait()

    return sc_kernel(P, x_flat)


def kernel(x, embedding, W, b):
    B, L = x.shape
    D = W.shape[1]
    P = _project_table(embedding, W, b)
    x_flat = x.reshape(B * L).astype(jnp.int32)
    out_t = _gather_normalize(P, x_flat, B, L, D)
    return out_t.transpose(1, 0, 2)
